# baseline (speedup 1.0000x reference)
import functools
import jax
import jax.numpy as jnp
from jax import lax
from jax.experimental import pallas as pl
from jax.experimental.pallas import tpu as pltpu
from jax.experimental.pallas import tpu_sc as plsc

KS = 5
K = 25
G = 64
N = 50000
E = 800000
NW = 32
NPT = 1568
NP = NW * NPT
EP = 802816
CH1 = 2048
NCH1 = EP // CH1
CH2 = 1024
NCH2 = EP // CH2
RB = 1536
FLUSH = RB - CH2
GB = 128
TCB = 512
NTCB = NP // TCB

_mesh = plsc.VectorSubcoreMesh(core_axis_name="c", subcore_axis_name="s")
_sc_params = pltpu.CompilerParams(needs_layout_passes=False,
                                  use_tc_tiling_on_sc=False)


def _wid():
    return lax.axis_index("s") * 2 + lax.axis_index("c")


def _zero_f32(ref, nwords):
    z16 = jnp.zeros((16,), jnp.float32)

    def body(i, _):
        ref[pl.ds(i * 16, 16)] = z16
        return 0

    lax.fori_loop(0, nwords // 16, body, 0)


def _taps(a0, a1):
    v0 = a0 * (KS - 1.0)
    v1 = a1 * (KS - 1.0)
    b0 = jnp.clip(v0.astype(jnp.int32), 0, KS - 2)
    b1 = jnp.clip(v1.astype(jnp.int32), 0, KS - 2)
    f0 = v0 - b0.astype(jnp.float32)
    f1 = v1 - b1.astype(jnp.float32)
    w = ((1.0 - f0) * (1.0 - f1), f0 * (1.0 - f1), (1.0 - f0) * f1, f0 * f1)
    tap0 = b0 + b1 * KS
    return tap0, w


_TAP_OFF = (0, 1, KS, KS + 1)


def _edge_fields(ebuf, i):
    s16 = ebuf[0, pl.ds(i * 16, 16)]
    d16 = ebuf[1, pl.ds(i * 16, 16)]
    a0 = lax.bitcast_convert_type(ebuf[2, pl.ds(i * 16, 16)], jnp.float32)
    a1 = lax.bitcast_convert_type(ebuf[3, pl.ds(i * 16, 16)], jnp.float32)
    return s16, d16, a0, a1


def _sc1_body(ep_hbm, x_hbm, z1_hbm, deg_hbm, xv, ebuf, z1acc, degacc):
    wid = _wid()
    base = wid * NPT
    pltpu.sync_copy(x_hbm, xv)
    _zero_f32(z1acc, NPT * K)
    _zero_f32(degacc, NPT)
    ones16 = jnp.ones((16,), jnp.float32)

    def chunk(c, _):
        pltpu.sync_copy(ep_hbm.at[c], ebuf)

        def vec(i, _):
            s16, d16, a0, a1 = _edge_fields(ebuf, i)
            tap0, w = _taps(a0, a1)
            mask = (d16 >= base) & (d16 < base + NPT)
            dl = jnp.where(mask, d16 - base, 0)
            xj = plsc.load_gather(xv, [s16])
            tbase = dl * K + tap0
            for t in range(4):
                plsc.addupdate_scatter(z1acc, [tbase + _TAP_OFF[t]], w[t] * xj,
                                       mask=mask)
            plsc.addupdate_scatter(degacc, [dl], ones16, mask=mask)
            return 0

        lax.fori_loop(0, CH1 // 16, vec, 0)
        return 0

    lax.fori_loop(0, NCH1, chunk, 0)
    pltpu.sync_copy(z1acc, z1_hbm.at[wid])
    pltpu.sync_copy(degacc, deg_hbm.at[wid])


@functools.partial(
    pl.kernel, mesh=_mesh,
    out_type=[
        jax.ShapeDtypeStruct((NW, NPT * K), jnp.float32),
        jax.ShapeDtypeStruct((NW, NPT), jnp.float32),
    ],
    scratch_types=[
        pltpu.VMEM((NP,), jnp.float32),
        pltpu.VMEM((4, CH1), jnp.int32),
        pltpu.VMEM((NPT * K,), jnp.float32),
        pltpu.VMEM((NPT,), jnp.float32),
    ],
    compiler_params=_sc_params,
)
def _sc1(*refs):
    _sc1_body(*refs)


def _sc2_body(ep_hbm, h2_hbm, agg_hbm, ebuf, idx4, w4, dloc, rows, acc, sem):
    wid = _wid()
    base = wid * NPT
    _zero_f32(acc, NPT * 64)
    z16f = jnp.zeros((16,), jnp.float32)
    z16i = jnp.zeros((16,), jnp.int32)

    def zero_ring(_):
        def zb(i, __):
            w4[pl.ds(i * 16, 16)] = z16f
            return 0
        lax.fori_loop(0, 4 * RB // 16, zb, 0)
        def zi(i, __):
            idx4[pl.ds(i * 16, 16)] = z16i
            return 0
        lax.fori_loop(0, 4 * RB // 16, zi, 0)
        def zd(i, __):
            dloc[pl.ds(i * 16, 16)] = z16i
            return 0
        lax.fori_loop(0, RB // 16, zd, 0)
        return 0

    zero_ring(0)
    lanes = lax.iota(jnp.int32, 16)

    def flush(cnt):
        nb = (cnt + GB - 1) // GB

        def gblk(g, _):
            off = g * GB
            for t in range(4):
                pltpu.async_copy(h2_hbm.at[idx4.at[pl.ds(t * RB + off, GB)]],
                                 rows, sem).wait()

                def grp(q, __):
                    w16 = w4[pl.ds(t * RB + off + q * 16, 16)]
                    dl16 = dloc[pl.ds(off + q * 16, 16)]
                    tgt0 = dl16 * 64
                    r16 = q * 16 + lanes

                    def feat(fo, ___):
                        for fu in range(8):
                            f = fo * 8 + fu
                            vals = plsc.load_gather(rows, [r16, z16i + f])
                            plsc.addupdate_scatter(acc, [tgt0 + f], w16 * vals)
                        return 0

                    lax.fori_loop(0, 8, feat, 0)
                    return 0

                lax.fori_loop(0, GB // 16, grp, 0)
            return 0

        lax.fori_loop(0, nb, gblk, 0)

    def chunk(c, cnt):
        pltpu.sync_copy(ep_hbm.at[c], ebuf)

        def vec(i, cn):
            s16, d16, a0, a1 = _edge_fields(ebuf, i)
            tap0, w = _taps(a0, a1)
            mask = (d16 >= base) & (d16 < base + NPT)
            dl = jnp.where(mask, d16 - base, 0)
            rbase = s16 * K + tap0
            for t in range(4):
                plsc.store_compressed(idx4.at[pl.ds(t * RB + cn, 16)],
                                      rbase + _TAP_OFF[t], mask=mask)
                plsc.store_compressed(w4.at[pl.ds(t * RB + cn, 16)], w[t],
                                      mask=mask)
            plsc.store_compressed(dloc.at[pl.ds(cn, 16)], dl, mask=mask)
            return cn + jnp.sum(mask.astype(jnp.int32))

        cnt = lax.fori_loop(0, CH2 // 16, vec, cnt)

        def do_flush(cn):
            flush(cn)
            zero_ring(0)
            return 0

        cnt = lax.cond(cnt >= FLUSH, do_flush, lambda cn: cn, cnt)
        return cnt

    cnt = lax.fori_loop(0, NCH2, chunk, 0)
    flush(cnt)
    pltpu.sync_copy(acc, agg_hbm.at[wid])


@functools.partial(
    pl.kernel, mesh=_mesh,
    out_type=jax.ShapeDtypeStruct((NW, NPT * 64), jnp.float32),
    scratch_types=[
        pltpu.VMEM((4, CH2), jnp.int32),
        pltpu.VMEM((4 * RB,), jnp.int32),
        pltpu.VMEM((4 * RB,), jnp.float32),
        pltpu.VMEM((RB,), jnp.int32),
        pltpu.VMEM((GB, 64), jnp.float32),
        pltpu.VMEM((NPT * 64,), jnp.float32),
        pltpu.SemaphoreType.DMA,
    ],
    compiler_params=_sc_params,
)
def _sc2(*refs):
    _sc2_body(*refs)


def _elu(t):
    return jnp.where(t > 0.0, t, jnp.exp(jnp.minimum(t, 0.0)) - 1.0)


def _tc1_kernel(z1_ref, deg_ref, x_ref, w1_ref, r1_ref, b1_ref, w2_ref,
                r2_ref, h2_ref, hr2_ref):
    z1 = z1_ref[...]
    deg = jnp.maximum(deg_ref[...], 1.0)
    h1 = z1 @ w1_ref[...] / deg + x_ref[...] @ r1_ref[...] + b1_ref[...]
    h1 = _elu(h1)
    h2_ref[...] = h1 @ w2_ref[...]
    hr2_ref[...] = h1 @ r2_ref[...]


def _tc2_kernel(agg_ref, deg_ref, hr2_ref, b2_ref, p_ref, wl1_ref, bl1_ref,
                wl2_ref, bl2_ref, out_ref, pool_ref, cnt_ref):
    j = pl.program_id(0)

    @pl.when(j == 0)
    def _init():
        pool_ref[...] = jnp.zeros_like(pool_ref)
        cnt_ref[...] = jnp.zeros_like(cnt_ref)

    deg = jnp.maximum(deg_ref[...], 1.0)
    h2 = _elu(agg_ref[...] / deg + hr2_ref[...] + b2_ref[...])
    p = p_ref[...]
    pool_ref[...] += p @ h2
    cnt_ref[...] += p @ jnp.ones_like(h2)

    @pl.when(j == NTCB - 1)
    def _head():
        pooled = pool_ref[...] / jnp.maximum(cnt_ref[...], 1.0)
        t1 = _elu(pooled @ wl1_ref[...] + bl1_ref[...])
        logits = t1 @ wl2_ref[...] + bl2_ref[...]
        m = jnp.max(logits, axis=1, keepdims=True)
        s = logits - m
        lse = jnp.log(jnp.sum(jnp.exp(s), axis=1, keepdims=True))
        out_ref[...] = s - lse


def _row_spec(cols):
    return pl.BlockSpec((TCB, cols), lambda j: (j, 0))


def _full_spec(r, c):
    return pl.BlockSpec((r, c), lambda j: (0, 0))


def kernel(x, edge_index, edge_attr, batch, W1, root1, b1, W2, root2, b2,
           Wl1, bl1, Wl2, bl2):
    src = edge_index[0].astype(jnp.int32)
    dst = edge_index[1].astype(jnp.int32)
    pad_e = EP - E
    src_p = jnp.concatenate([src, jnp.zeros((pad_e,), jnp.int32)])
    dst_p = jnp.concatenate([dst, jnp.full((pad_e,), NP, jnp.int32)])
    a0_p = jnp.concatenate([edge_attr[:, 0], jnp.zeros((pad_e,), jnp.float32)])
    a1_p = jnp.concatenate([edge_attr[:, 1], jnp.zeros((pad_e,), jnp.float32)])
    stacked = jnp.stack([
        src_p, dst_p,
        lax.bitcast_convert_type(a0_p, jnp.int32),
        lax.bitcast_convert_type(a1_p, jnp.int32),
    ])
    ep1 = stacked.reshape(4, NCH1, CH1).transpose(1, 0, 2)
    ep2 = stacked.reshape(4, NCH2, CH2).transpose(1, 0, 2)
    x_p = jnp.concatenate([x[:, 0], jnp.zeros((NP - N,), jnp.float32)])

    z1_2d, deg_2d = _sc1(ep1, x_p)
    z1 = z1_2d.reshape(NP, K)
    deg = deg_2d.reshape(NP, 1)

    w1m = W1.reshape(K, 32)
    w2m = W2.transpose(1, 0, 2).reshape(32, K * 64)
    h2rows, hr2 = pl.pallas_call(
        _tc1_kernel,
        grid=(NTCB,),
        in_specs=[
            _row_spec(K), _row_spec(1), _row_spec(1),
            _full_spec(K, 32), _full_spec(1, 32), _full_spec(1, 32),
            _full_spec(32, K * 64), _full_spec(32, 64),
        ],
        out_specs=[_row_spec(K * 64), _row_spec(64)],
        out_shape=[
            jax.ShapeDtypeStruct((NP, K * 64), jnp.float32),
            jax.ShapeDtypeStruct((NP, 64), jnp.float32),
        ],
    )(z1, deg, x_p[:, None], w1m, root1, b1[None, :], w2m, root2)

    agg_2d = _sc2(ep2, h2rows.reshape(NP * K, 64))
    agg = agg_2d.reshape(NP, 64)

    batch_p = jnp.concatenate([batch.astype(jnp.int32),
                               jnp.full((NP - N,), G, jnp.int32)])
    onehot = (batch_p[None, :] == jnp.arange(G, dtype=jnp.int32)[:, None])
    onehot = onehot.astype(jnp.float32)

    return pl.pallas_call(
        _tc2_kernel,
        grid=(NTCB,),
        in_specs=[
            _row_spec(64), _row_spec(1), _row_spec(64),
            _full_spec(1, 64),
            pl.BlockSpec((G, TCB), lambda j: (0, j)),
            _full_spec(64, 128), _full_spec(1, 128),
            _full_spec(128, 10), _full_spec(1, 10),
        ],
        out_specs=pl.BlockSpec((G, 10), lambda j: (0, 0)),
        out_shape=jax.ShapeDtypeStruct((G, 10), jnp.float32),
        scratch_shapes=[
            pltpu.VMEM((G, 64), jnp.float32),
            pltpu.VMEM((G, 64), jnp.float32),
        ],
    )(agg, deg, hr2, b2[None, :], onehot, Wl1, bl1[None, :], Wl2, bl2[None, :])


# speedup vs baseline: 1.0052x; 1.0052x over previous
import functools
import jax
import jax.numpy as jnp
from jax import lax
from jax.experimental import pallas as pl
from jax.experimental.pallas import tpu as pltpu
from jax.experimental.pallas import tpu_sc as plsc

KS = 5
K = 25
G = 64
N = 50000
E = 800000
NW = 32
NPT = 1568
NP = NW * NPT
EP = 802816
CH1 = 2048
NCH1 = EP // CH1
CH2 = 512
NCH2 = EP // CH2
RB = 1024
FLUSH = RB - CH2
GB = 128
TCB = 512
NTCB = NP // TCB

_mesh = plsc.VectorSubcoreMesh(core_axis_name="c", subcore_axis_name="s")
_sc_params = pltpu.CompilerParams(needs_layout_passes=False,
                                  use_tc_tiling_on_sc=False)


def _wid():
    return lax.axis_index("s") * 2 + lax.axis_index("c")


def _zero_f32(ref, nwords):
    z16 = jnp.zeros((16,), jnp.float32)

    def body(i, _):
        ref[pl.ds(i * 16, 16)] = z16
        return 0

    lax.fori_loop(0, nwords // 16, body, 0)


def _taps(a0, a1):
    v0 = a0 * (KS - 1.0)
    v1 = a1 * (KS - 1.0)
    b0 = jnp.clip(v0.astype(jnp.int32), 0, KS - 2)
    b1 = jnp.clip(v1.astype(jnp.int32), 0, KS - 2)
    f0 = v0 - b0.astype(jnp.float32)
    f1 = v1 - b1.astype(jnp.float32)
    w = ((1.0 - f0) * (1.0 - f1), f0 * (1.0 - f1), (1.0 - f0) * f1, f0 * f1)
    tap0 = b0 + b1 * KS
    return tap0, w


_TAP_OFF = (0, 1, KS, KS + 1)


def _edge_fields(ebuf, i):
    s16 = ebuf[0, pl.ds(i * 16, 16)]
    d16 = ebuf[1, pl.ds(i * 16, 16)]
    a0 = lax.bitcast_convert_type(ebuf[2, pl.ds(i * 16, 16)], jnp.float32)
    a1 = lax.bitcast_convert_type(ebuf[3, pl.ds(i * 16, 16)], jnp.float32)
    return s16, d16, a0, a1


def _sc1_body(ep_hbm, x_hbm, z1_hbm, deg_hbm, xv, ebuf, z1acc, degacc):
    wid = _wid()
    base = wid * NPT
    pltpu.sync_copy(x_hbm, xv)
    _zero_f32(z1acc, NPT * K)
    _zero_f32(degacc, NPT)
    ones16 = jnp.ones((16,), jnp.float32)

    def chunk(c, _):
        pltpu.sync_copy(ep_hbm.at[c], ebuf)

        def vec(i, _):
            s16, d16, a0, a1 = _edge_fields(ebuf, i)
            tap0, w = _taps(a0, a1)
            mask = (d16 >= base) & (d16 < base + NPT)
            dl = jnp.where(mask, d16 - base, 0)
            xj = plsc.load_gather(xv, [s16])
            tbase = dl * K + tap0
            for t in range(4):
                plsc.addupdate_scatter(z1acc, [tbase + _TAP_OFF[t]], w[t] * xj,
                                       mask=mask)
            plsc.addupdate_scatter(degacc, [dl], ones16, mask=mask)
            return 0

        lax.fori_loop(0, CH1 // 16, vec, 0)
        return 0

    lax.fori_loop(0, NCH1, chunk, 0)
    pltpu.sync_copy(z1acc, z1_hbm.at[wid])
    pltpu.sync_copy(degacc, deg_hbm.at[wid])


@functools.partial(
    pl.kernel, mesh=_mesh,
    out_type=[
        jax.ShapeDtypeStruct((NW, NPT * K), jnp.float32),
        jax.ShapeDtypeStruct((NW, NPT), jnp.float32),
    ],
    scratch_types=[
        pltpu.VMEM((NP,), jnp.float32),
        pltpu.VMEM((4, CH1), jnp.int32),
        pltpu.VMEM((NPT * K,), jnp.float32),
        pltpu.VMEM((NPT,), jnp.float32),
    ],
    compiler_params=_sc_params,
)
def _sc1(*refs):
    _sc1_body(*refs)


def _sc2_body(ep_hbm, h2_hbm, agg_hbm, ebuf, idx4, w4, dloc, rows, acc,
              sem0, sem1):
    wid = _wid()
    base = wid * NPT
    _zero_f32(acc, NPT * 64)
    z16f = jnp.zeros((16,), jnp.float32)
    z16i = jnp.zeros((16,), jnp.int32)

    def zero_ring(_):
        def zb(i, __):
            w4[pl.ds(i * 16, 16)] = z16f
            return 0
        lax.fori_loop(0, 4 * RB // 16, zb, 0)
        def zi(i, __):
            idx4[pl.ds(i * 16, 16)] = z16i
            return 0
        lax.fori_loop(0, 4 * RB // 16, zi, 0)
        def zd(i, __):
            dloc[pl.ds(i * 16, 16)] = z16i
            return 0
        lax.fori_loop(0, RB // 16, zd, 0)
        return 0

    zero_ring(0)
    lanes = lax.iota(jnp.int32, 16)

    def flush(cnt):
        nb = (cnt + GB - 1) // GB
        nu = nb * 4
        r0 = rows.at[0]
        r1 = rows.at[1]

        def ring_off(u):
            return (u % 4) * RB + (u // 4) * GB

        def fire(u, rbuf, sm):
            pltpu.async_copy(h2_hbm.at[idx4.at[pl.ds(ring_off(u), GB)]],
                             rbuf, sm)

        def drain(u, rbuf, sm):
            pltpu.make_async_copy(
                h2_hbm.at[idx4.at[pl.ds(ring_off(u), GB)]], rbuf, sm).wait()

        def accum(u, rbuf):
            off = ring_off(u)
            doff = (u // 4) * GB

            def grp(q, __):
                w16 = w4[pl.ds(off + q * 16, 16)]
                dl16 = dloc[pl.ds(doff + q * 16, 16)]
                tgt0 = dl16 * 64
                r16 = q * 16 + lanes

                def feat(fo, ___):
                    for fu in range(8):
                        f = fo * 8 + fu
                        vals = plsc.load_gather(rbuf, [r16, z16i + f])
                        plsc.addupdate_scatter(acc, [tgt0 + f], w16 * vals)
                    return 0

                lax.fori_loop(0, 8, feat, 0)
                return 0

            lax.fori_loop(0, GB // 16, grp, 0)

        @pl.when(nu > 0)
        def _pipe():
            fire(0, r0, sem0)

            def body(j, _):
                u0 = 2 * j
                u1 = u0 + 1
                fire(u1, r1, sem1)
                drain(u0, r0, sem0)
                accum(u0, r0)

                @pl.when(u1 + 1 < nu)
                def _next():
                    fire(u1 + 1, r0, sem0)

                drain(u1, r1, sem1)
                accum(u1, r1)
                return 0

            lax.fori_loop(0, nu // 2, body, 0)

    def chunk(c, cnt):
        pltpu.sync_copy(ep_hbm.at[c], ebuf)

        def vec(i, cn):
            s16, d16, a0, a1 = _edge_fields(ebuf, i)
            tap0, w = _taps(a0, a1)
            mask = (d16 >= base) & (d16 < base + NPT)
            dl = jnp.where(mask, d16 - base, 0)
            rbase = s16 * K + tap0
            for t in range(4):
                plsc.store_compressed(idx4.at[pl.ds(t * RB + cn, 16)],
                                      rbase + _TAP_OFF[t], mask=mask)
                plsc.store_compressed(w4.at[pl.ds(t * RB + cn, 16)], w[t],
                                      mask=mask)
            plsc.store_compressed(dloc.at[pl.ds(cn, 16)], dl, mask=mask)
            return cn + jnp.sum(mask.astype(jnp.int32))

        cnt = lax.fori_loop(0, CH2 // 16, vec, cnt)

        def do_flush(cn):
            flush(cn)
            zero_ring(0)
            return 0

        cnt = lax.cond(cnt >= FLUSH, do_flush, lambda cn: cn, cnt)
        return cnt

    cnt = lax.fori_loop(0, NCH2, chunk, 0)
    flush(cnt)
    pltpu.sync_copy(acc, agg_hbm.at[wid])


@functools.partial(
    pl.kernel, mesh=_mesh,
    out_type=jax.ShapeDtypeStruct((NW, NPT * 64), jnp.float32),
    scratch_types=[
        pltpu.VMEM((4, CH2), jnp.int32),
        pltpu.VMEM((4 * RB,), jnp.int32),
        pltpu.VMEM((4 * RB,), jnp.float32),
        pltpu.VMEM((RB,), jnp.int32),
        pltpu.VMEM((2, GB, 64), jnp.float32),
        pltpu.VMEM((NPT * 64,), jnp.float32),
        pltpu.SemaphoreType.DMA,
        pltpu.SemaphoreType.DMA,
    ],
    compiler_params=_sc_params,
)
def _sc2(*refs):
    _sc2_body(*refs)


def _elu(t):
    return jnp.where(t > 0.0, t, jnp.exp(jnp.minimum(t, 0.0)) - 1.0)


def _tc1_kernel(z1_ref, deg_ref, x_ref, w1_ref, r1_ref, b1_ref, w2_ref,
                r2_ref, h2_ref, hr2_ref):
    z1 = z1_ref[...]
    deg = jnp.maximum(deg_ref[...], 1.0)
    h1 = z1 @ w1_ref[...] / deg + x_ref[...] @ r1_ref[...] + b1_ref[...]
    h1 = _elu(h1)
    h2_ref[...] = h1 @ w2_ref[...]
    hr2_ref[...] = h1 @ r2_ref[...]


def _tc2_kernel(agg_ref, deg_ref, hr2_ref, b2_ref, p_ref, wl1_ref, bl1_ref,
                wl2_ref, bl2_ref, out_ref, pool_ref, cnt_ref):
    j = pl.program_id(0)

    @pl.when(j == 0)
    def _init():
        pool_ref[...] = jnp.zeros_like(pool_ref)
        cnt_ref[...] = jnp.zeros_like(cnt_ref)

    deg = jnp.maximum(deg_ref[...], 1.0)
    h2 = _elu(agg_ref[...] / deg + hr2_ref[...] + b2_ref[...])
    p = p_ref[...]
    pool_ref[...] += p @ h2
    cnt_ref[...] += p @ jnp.ones_like(h2)

    @pl.when(j == NTCB - 1)
    def _head():
        pooled = pool_ref[...] / jnp.maximum(cnt_ref[...], 1.0)
        t1 = _elu(pooled @ wl1_ref[...] + bl1_ref[...])
        logits = t1 @ wl2_ref[...] + bl2_ref[...]
        m = jnp.max(logits, axis=1, keepdims=True)
        s = logits - m
        lse = jnp.log(jnp.sum(jnp.exp(s), axis=1, keepdims=True))
        out_ref[...] = s - lse


def _row_spec(cols):
    return pl.BlockSpec((TCB, cols), lambda j: (j, 0))


def _full_spec(r, c):
    return pl.BlockSpec((r, c), lambda j: (0, 0))


def kernel(x, edge_index, edge_attr, batch, W1, root1, b1, W2, root2, b2,
           Wl1, bl1, Wl2, bl2):
    src = edge_index[0].astype(jnp.int32)
    dst = edge_index[1].astype(jnp.int32)
    pad_e = EP - E
    src_p = jnp.concatenate([src, jnp.zeros((pad_e,), jnp.int32)])
    dst_p = jnp.concatenate([dst, jnp.full((pad_e,), NP, jnp.int32)])
    a0_p = jnp.concatenate([edge_attr[:, 0], jnp.zeros((pad_e,), jnp.float32)])
    a1_p = jnp.concatenate([edge_attr[:, 1], jnp.zeros((pad_e,), jnp.float32)])
    stacked = jnp.stack([
        src_p, dst_p,
        lax.bitcast_convert_type(a0_p, jnp.int32),
        lax.bitcast_convert_type(a1_p, jnp.int32),
    ])
    ep1 = stacked.reshape(4, NCH1, CH1).transpose(1, 0, 2)
    ep2 = stacked.reshape(4, NCH2, CH2).transpose(1, 0, 2)
    x_p = jnp.concatenate([x[:, 0], jnp.zeros((NP - N,), jnp.float32)])

    z1_2d, deg_2d = _sc1(ep1, x_p)
    z1 = z1_2d.reshape(NP, K)
    deg = deg_2d.reshape(NP, 1)

    w1m = W1.reshape(K, 32)
    w2m = W2.transpose(1, 0, 2).reshape(32, K * 64)
    h2rows, hr2 = pl.pallas_call(
        _tc1_kernel,
        grid=(NTCB,),
        in_specs=[
            _row_spec(K), _row_spec(1), _row_spec(1),
            _full_spec(K, 32), _full_spec(1, 32), _full_spec(1, 32),
            _full_spec(32, K * 64), _full_spec(32, 64),
        ],
        out_specs=[_row_spec(K * 64), _row_spec(64)],
        out_shape=[
            jax.ShapeDtypeStruct((NP, K * 64), jnp.float32),
            jax.ShapeDtypeStruct((NP, 64), jnp.float32),
        ],
    )(z1, deg, x_p[:, None], w1m, root1, b1[None, :], w2m, root2)

    agg_2d = _sc2(ep2, h2rows.reshape(NP * K, 64))
    agg = agg_2d.reshape(NP, 64)

    batch_p = jnp.concatenate([batch.astype(jnp.int32),
                               jnp.full((NP - N,), G, jnp.int32)])
    onehot = (batch_p[None, :] == jnp.arange(G, dtype=jnp.int32)[:, None])
    onehot = onehot.astype(jnp.float32)

    return pl.pallas_call(
        _tc2_kernel,
        grid=(NTCB,),
        in_specs=[
            _row_spec(64), _row_spec(1), _row_spec(64),
            _full_spec(1, 64),
            pl.BlockSpec((G, TCB), lambda j: (0, j)),
            _full_spec(64, 128), _full_spec(1, 128),
            _full_spec(128, 10), _full_spec(1, 10),
        ],
        out_specs=pl.BlockSpec((G, 10), lambda j: (0, 0)),
        out_shape=jax.ShapeDtypeStruct((G, 10), jnp.float32),
        scratch_shapes=[
            pltpu.VMEM((G, 64), jnp.float32),
            pltpu.VMEM((G, 64), jnp.float32),
        ],
    )(agg, deg, hr2, b2[None, :], onehot, Wl1, bl1[None, :], Wl2, bl2[None, :])
